# Initial kernel scaffold; baseline (speedup 1.0000x reference)
import jax
import jax.numpy as jnp
from jax.experimental import pallas as pl


def kernel(title_emb_mat, emp_ids, end_yrs, batch_label, inputs, ref_batch_pos, ref_job_idx, ref_title_idx):
    raise NotImplementedError("write your pallas kernel here")


# speedup vs baseline: 8.0675x; 8.0675x over previous
import functools

import jax
import jax.numpy as jnp
from jax import lax
from jax.experimental import pallas as pl
from jax.experimental.pallas import tpu as pltpu
from jax.experimental.pallas import tpu_sc as plsc

_NC = 2
_NS = 16
_NW = _NC * _NS
_L = 16


def _lane_sum(x):
    for m in (1, 2, 4, 8):
        perm = lax.iota(jnp.int32, _L) ^ m
        x = x + jnp.take_along_axis(x, perm, axis=0)
    return x


def _build_sc_call(N, D, Td, P, R, C):
    rows_per_w = N // _NW
    CH = rows_per_w // C
    KD = D // _L
    KT = Td // _L
    G = C * R
    NIDX = G // 128

    mesh = plsc.VectorSubcoreMesh(core_axis_name="c", subcore_axis_name="s")

    @functools.partial(
        pl.kernel,
        mesh=mesh,
        out_type=jax.ShapeDtypeStruct((N, D + Td), jnp.float32),
        compiler_params=pltpu.CompilerParams(needs_layout_passes=False),
        scratch_types=[
            pltpu.VMEM((G,), jnp.int32),
            pltpu.VMEM((G,), jnp.int32),
            pltpu.VMEM((NIDX, 128), jnp.int32),
            pltpu.VMEM((NIDX, 128), jnp.int32),
            pltpu.VMEM((C, D), jnp.float32),
            pltpu.VMEM((G, D), jnp.float32),
            pltpu.VMEM((G, Td), jnp.float32),
            pltpu.VMEM((R * C,), jnp.float32),
            pltpu.VMEM((R * C,), jnp.float32),
            pltpu.VMEM((C,), jnp.int32),
            pltpu.VMEM((C, D + Td), jnp.float32),
            pltpu.SemaphoreType.DMA,
        ],
    )
    def sc_fn(inp_hbm, temb_hbm, pos_hbm, job_hbm, tit_hbm, ey_hbm, out_hbm,
              pos_v, job_v, fidx_v, tidx_v, focal_v, ref_v, trow_v,
              s_v, sim_v, ey_v, out_v, sem):
        wid = lax.axis_index("s") * _NC + lax.axis_index("c")
        base0 = wid * rows_per_w

        def chunk_body(ci, carry):
            base = base0 + ci * C
            b8 = base * R
            pltpu.sync_copy(pos_hbm.at[pl.ds(b8, G)], pos_v)
            pltpu.sync_copy(job_hbm.at[pl.ds(b8, G)], job_v)
            for j in range(NIDX):
                pltpu.sync_copy(tit_hbm.at[pl.ds(b8 + j * 128, 128)],
                                tidx_v.at[j])
            pltpu.sync_copy(ey_hbm.at[pl.ds(base, C)], ey_v)

            for k in range(G // _L):
                j = (k * _L) // 128
                col = (k * _L) % 128
                f = pos_v[pl.ds(k * _L, _L)] * P + job_v[pl.ds(k * _L, _L)]
                fidx_v[j, pl.ds(col, _L)] = f

            cps = []
            for j in range(NIDX):
                cps.append(pltpu.async_copy(
                    inp_hbm.at[fidx_v.at[j]],
                    ref_v.at[pl.ds(j * 128, 128)], sem))
                cps.append(pltpu.async_copy(
                    temb_hbm.at[tidx_v.at[j]],
                    trow_v.at[pl.ds(j * 128, 128)], sem))
            pltpu.sync_copy(inp_hbm.at[pl.ds(base, C)], focal_v)
            for cp in cps:
                cp.wait()

            lane0 = lax.iota(jnp.int32, _L) == 0

            def dot_body(n, c2):
                j0 = n * R
                fv = [focal_v[n, pl.ds(k * _L, _L)] for k in range(KD)]
                for r in range(R):
                    acc = fv[0] * ref_v[j0 + r, pl.ds(0, _L)]
                    for k in range(1, KD):
                        acc = acc + fv[k] * ref_v[j0 + r, pl.ds(k * _L, _L)]
                    dv = _lane_sum(acc)
                    idx = jnp.full((_L,), r * C, jnp.int32) + n
                    plsc.store_scatter(s_v, [idx], dv, mask=lane0)
                return c2
            lax.fori_loop(0, C, dot_body, 0)

            for v in range(C // _L):
                n0 = v * _L
                eys = ey_v[pl.ds(n0, _L)]
                es = [jnp.exp(s_v[pl.ds(r * C + n0, _L)]) for r in range(R)]
                den = es[0]
                for r in range(1, R):
                    den = den + es[r]
                inv = 1.0 / den
                keep = eys != 0
                zero = jnp.zeros((_L,), jnp.float32)
                for r in range(R):
                    sim_v[pl.ds(r * C + n0, _L)] = jnp.where(keep,
                                                             es[r] * inv,
                                                             zero)

            def out_body(n, c2):
                j0 = n * R
                ws = [plsc.load_gather(
                          sim_v, [jnp.full((_L,), r * C, jnp.int32) + n])
                      for r in range(R)]
                for k in range(KT):
                    acc = ws[0] * trow_v[j0, pl.ds(k * _L, _L)]
                    for r in range(1, R):
                        acc = acc + ws[r] * trow_v[j0 + r, pl.ds(k * _L, _L)]
                    out_v[n, pl.ds(D + k * _L, _L)] = acc
                for k in range(KD):
                    out_v[n, pl.ds(k * _L, _L)] = focal_v[n, pl.ds(k * _L, _L)]
                return c2
            lax.fori_loop(0, C, out_body, 0)

            pltpu.sync_copy(out_v, out_hbm.at[pl.ds(base, C)])
            return carry

        lax.fori_loop(0, CH, chunk_body, 0)

    return sc_fn


def kernel(title_emb_mat, emp_ids, end_yrs, batch_label, inputs,
           ref_batch_pos, ref_job_idx, ref_title_idx):
    B, P, D = inputs.shape
    T, Td = title_emb_mat.shape
    R = ref_batch_pos.shape[-1]
    N = B * P

    inp_flat = inputs.reshape(N, D)
    pos_flat = ref_batch_pos.astype(jnp.int32).reshape(-1)
    job_flat = ref_job_idx.astype(jnp.int32).reshape(-1)
    tit_flat = ref_title_idx.astype(jnp.int32).reshape(-1)
    ey_flat = end_yrs.astype(jnp.int32).reshape(-1)

    fn = _build_sc_call(N, D, Td, P, R, C=32)
    return fn(inp_flat, title_emb_mat, pos_flat, job_flat, tit_flat, ey_flat)


# speedup vs baseline: 11.9860x; 1.4857x over previous
import functools

import jax
import jax.numpy as jnp
from jax import lax
from jax.experimental import pallas as pl
from jax.experimental.pallas import tpu as pltpu
from jax.experimental.pallas import tpu_sc as plsc

_NC = 2
_NS = 16
_NW = _NC * _NS
_L = 16


def _lane_sum(x):
    for m in (1, 2, 4, 8):
        perm = lax.iota(jnp.int32, _L) ^ m
        x = x + jnp.take_along_axis(x, perm, axis=0)
    return x


def _tree_sum(terms):
    while len(terms) > 1:
        nxt = [terms[i] + terms[i + 1] for i in range(0, len(terms) - 1, 2)]
        if len(terms) % 2:
            nxt.append(terms[-1])
        terms = nxt
    return terms[0]


def _build_sc_call(N, D, Td, P, R, C):
    rows_per_w = N // _NW
    CH = rows_per_w // C
    KD = D // _L
    KT = Td // _L
    G = C * R
    assert G == 128
    IW = rows_per_w * R // 128

    mesh = plsc.VectorSubcoreMesh(core_axis_name="c", subcore_axis_name="s")

    @functools.partial(
        pl.kernel,
        mesh=mesh,
        out_type=jax.ShapeDtypeStruct((N, D + Td), jnp.float32),
        compiler_params=pltpu.CompilerParams(needs_layout_passes=False),
        scratch_types=[
            pltpu.VMEM((IW, 128), jnp.int32),
            pltpu.VMEM((IW, 128), jnp.int32),
            pltpu.VMEM((IW, 128), jnp.int32),
            pltpu.VMEM((IW, 128), jnp.int32),
            pltpu.VMEM((2, C, D), jnp.float32),
            pltpu.VMEM((2, G, D), jnp.float32),
            pltpu.VMEM((2, G, Td), jnp.float32),
            pltpu.VMEM((G,), jnp.float32),
            pltpu.VMEM((G,), jnp.float32),
            pltpu.VMEM((rows_per_w,), jnp.int32),
            pltpu.VMEM((2, C, Td), jnp.float32),
            pltpu.SemaphoreType.DMA((2,)),
        ],
    )
    def sc_fn(inp_hbm, temb_hbm, pos_hbm, job_hbm, tit_hbm, ey_hbm, out_hbm,
              pos_v, job_v, fidx_v, tidx_v, focal_v, ref_v, trow_v,
              s_v, sim_v, ey_v, out_v, sem_in):
        wid = lax.axis_index("s") * _NC + lax.axis_index("c")
        base0 = wid * rows_per_w

        pltpu.sync_copy(pos_hbm.at[pl.ds(wid * IW, IW)], pos_v)
        pltpu.sync_copy(job_hbm.at[pl.ds(wid * IW, IW)], job_v)
        pltpu.sync_copy(tit_hbm.at[pl.ds(wid * IW, IW)], tidx_v)
        pltpu.sync_copy(ey_hbm.at[pl.ds(base0, rows_per_w)], ey_v)

        def flat_body(row, carry):
            for cc in range(128 // _L):
                f = (pos_v[row, pl.ds(cc * _L, _L)] * P
                     + job_v[row, pl.ds(cc * _L, _L)])
                fidx_v[row, pl.ds(cc * _L, _L)] = f
            return carry
        lax.fori_loop(0, IW, flat_body, 0)

        def issue(ci, p):
            base = base0 + ci * C
            pltpu.async_copy(inp_hbm.at[fidx_v.at[ci]], ref_v.at[p],
                             sem_in.at[p])
            pltpu.async_copy(temb_hbm.at[tidx_v.at[ci]], trow_v.at[p],
                             sem_in.at[p])
            pltpu.async_copy(inp_hbm.at[pl.ds(base, C)], focal_v.at[p],
                             sem_in.at[p])

        def drain(ci, p):
            base = base0 + ci * C
            pltpu.make_async_copy(inp_hbm.at[fidx_v.at[ci]], ref_v.at[p],
                                  sem_in.at[p]).wait()
            pltpu.make_async_copy(temb_hbm.at[tidx_v.at[ci]], trow_v.at[p],
                                  sem_in.at[p]).wait()
            pltpu.make_async_copy(inp_hbm.at[pl.ds(base, C)], focal_v.at[p],
                                  sem_in.at[p]).wait()

        issue(0, 0)
        lane0 = lax.iota(jnp.int32, _L) == 0

        def chunk_body(ci, carry):
            p = ci & 1
            base = base0 + ci * C

            @pl.when(ci + 1 < CH)
            def _():
                issue(ci + 1, 1 - p)

            drain(ci, p)

            def dot_body(n, c2):
                j0 = n * R
                fv = [focal_v[p, n, pl.ds(k * _L, _L)] for k in range(KD)]
                for r in range(R):
                    prods = [fv[k] * ref_v[p, j0 + r, pl.ds(k * _L, _L)]
                             for k in range(KD)]
                    dv = _lane_sum(_tree_sum(prods))
                    idx = jnp.full((_L,), r * C, jnp.int32) + n
                    plsc.store_scatter(s_v, [idx], dv, mask=lane0)
                return c2
            lax.fori_loop(0, C, dot_body, 0)

            eys = ey_v[pl.ds(ci * C, C)]
            es = [jnp.exp(s_v[pl.ds(r * C, C)]) for r in range(R)]
            den = _tree_sum(es)
            inv = 1.0 / den
            keep = eys != 0
            zero = jnp.zeros((_L,), jnp.float32)
            for r in range(R):
                sim_v[pl.ds(r * C, C)] = jnp.where(keep, es[r] * inv, zero)

            def out_body(n, c2):
                j0 = n * R
                ws = [plsc.load_gather(
                          sim_v, [jnp.full((_L,), r * C, jnp.int32) + n])
                      for r in range(R)]
                for k in range(KT):
                    acc = _tree_sum(
                        [ws[r] * trow_v[p, j0 + r, pl.ds(k * _L, _L)]
                         for r in range(R)])
                    out_v[p, n, pl.ds(k * _L, _L)] = acc
                return c2
            lax.fori_loop(0, C, out_body, 0)

            pltpu.sync_copy(focal_v.at[p],
                            out_hbm.at[pl.ds(base, C), pl.ds(0, D)])
            pltpu.sync_copy(out_v.at[p],
                            out_hbm.at[pl.ds(base, C), pl.ds(D, Td)])
            return carry

        lax.fori_loop(0, CH, chunk_body, 0)

    return sc_fn


def kernel(title_emb_mat, emp_ids, end_yrs, batch_label, inputs,
           ref_batch_pos, ref_job_idx, ref_title_idx):
    B, P, D = inputs.shape
    T, Td = title_emb_mat.shape
    R = ref_batch_pos.shape[-1]
    N = B * P

    inp_flat = inputs.reshape(N, D)
    pos_rows = ref_batch_pos.astype(jnp.int32).reshape(-1, 128)
    job_rows = ref_job_idx.astype(jnp.int32).reshape(-1, 128)
    tit_rows = ref_title_idx.astype(jnp.int32).reshape(-1, 128)
    ey_flat = end_yrs.astype(jnp.int32).reshape(-1)

    fn = _build_sc_call(N, D, Td, P, R, C=128 // R)
    return fn(inp_flat, title_emb_mat, pos_rows, job_rows, tit_rows, ey_flat)


# speedup vs baseline: 14.5035x; 1.2100x over previous
import functools

import jax
import jax.numpy as jnp
from jax import lax
from jax.experimental import pallas as pl
from jax.experimental.pallas import tpu as pltpu
from jax.experimental.pallas import tpu_sc as plsc

_NC = 2
_NS = 16
_NW = _NC * _NS
_L = 16


def _lane_sum(x):
    for m in (1, 2, 4, 8):
        perm = lax.iota(jnp.int32, _L) ^ m
        x = x + jnp.take_along_axis(x, perm, axis=0)
    return x


def _tree_sum(terms):
    while len(terms) > 1:
        nxt = [terms[i] + terms[i + 1] for i in range(0, len(terms) - 1, 2)]
        if len(terms) % 2:
            nxt.append(terms[-1])
        terms = nxt
    return terms[0]


def _build_sc_call(N, D, Td, P, R, C):
    rows_per_w = N // _NW
    CH = rows_per_w // C
    KD = D // _L
    KT = Td // _L
    G = C * R
    assert G == 128
    IW = rows_per_w * R // 128

    mesh = plsc.VectorSubcoreMesh(core_axis_name="c", subcore_axis_name="s")

    @functools.partial(
        pl.kernel,
        mesh=mesh,
        out_type=jax.ShapeDtypeStruct((N, D + Td), jnp.float32),
        compiler_params=pltpu.CompilerParams(needs_layout_passes=False),
        scratch_types=[
            pltpu.VMEM((IW, 128), jnp.int32),
            pltpu.VMEM((IW, 128), jnp.int32),
            pltpu.VMEM((IW, 128), jnp.int32),
            pltpu.VMEM((IW, 128), jnp.int32),
            pltpu.VMEM((2, C, D), jnp.float32),
            pltpu.VMEM((2, G, D), jnp.float32),
            pltpu.VMEM((2, G, Td), jnp.float32),
            pltpu.VMEM((G,), jnp.float32),
            pltpu.VMEM((G,), jnp.float32),
            pltpu.VMEM((rows_per_w,), jnp.int32),
            pltpu.VMEM((2, C, Td), jnp.float32),
            pltpu.SemaphoreType.DMA((2,)),
        ],
    )
    def sc_fn(inp_hbm, temb_hbm, pos_hbm, job_hbm, tit_hbm, ey_hbm, out_hbm,
              pos_v, job_v, fidx_v, tidx_v, focal_v, ref_v, trow_v,
              s_v, sim_v, ey_v, out_v, sem_in):
        wid = lax.axis_index("s") * _NC + lax.axis_index("c")
        base0 = wid * rows_per_w

        pltpu.sync_copy(pos_hbm.at[pl.ds(wid * IW, IW)], pos_v)
        pltpu.sync_copy(job_hbm.at[pl.ds(wid * IW, IW)], job_v)
        pltpu.sync_copy(tit_hbm.at[pl.ds(wid * IW, IW)], tidx_v)
        pltpu.sync_copy(ey_hbm.at[pl.ds(base0, rows_per_w)], ey_v)

        def flat_body(row, carry):
            for cc in range(128 // _L):
                f = (pos_v[row, pl.ds(cc * _L, _L)] * P
                     + job_v[row, pl.ds(cc * _L, _L)])
                fidx_v[row, pl.ds(cc * _L, _L)] = f
            return carry
        lax.fori_loop(0, IW, flat_body, 0)

        def issue(ci, p):
            base = base0 + ci * C
            pltpu.async_copy(inp_hbm.at[fidx_v.at[ci]], ref_v.at[p],
                             sem_in.at[p])
            pltpu.async_copy(temb_hbm.at[tidx_v.at[ci]], trow_v.at[p],
                             sem_in.at[p])
            pltpu.async_copy(inp_hbm.at[pl.ds(base, C)], focal_v.at[p],
                             sem_in.at[p])

        def drain(ci, p):
            base = base0 + ci * C
            pltpu.make_async_copy(inp_hbm.at[fidx_v.at[ci]], ref_v.at[p],
                                  sem_in.at[p]).wait()
            pltpu.make_async_copy(temb_hbm.at[tidx_v.at[ci]], trow_v.at[p],
                                  sem_in.at[p]).wait()
            pltpu.make_async_copy(inp_hbm.at[pl.ds(base, C)], focal_v.at[p],
                                  sem_in.at[p]).wait()

        issue(0, 0)
        lane0 = lax.iota(jnp.int32, _L) == 0

        def chunk_body(ci, carry):
            p = ci & 1
            base = base0 + ci * C

            @pl.when(ci + 1 < CH)
            def _():
                issue(ci + 1, 1 - p)

            drain(ci, p)

            @plsc.parallel_loop(0, C, unroll=2)
            def dot_body(n):
                j0 = n * R
                fv = [focal_v[p, n, pl.ds(k * _L, _L)] for k in range(KD)]
                for r in range(R):
                    prods = [fv[k] * ref_v[p, j0 + r, pl.ds(k * _L, _L)]
                             for k in range(KD)]
                    dv = _lane_sum(_tree_sum(prods))
                    idx = jnp.full((_L,), r * C, jnp.int32) + n
                    plsc.store_scatter(s_v, [idx], dv, mask=lane0)

            eys = ey_v[pl.ds(ci * C, C)]
            es = [jnp.exp(s_v[pl.ds(r * C, C)]) for r in range(R)]
            den = _tree_sum(es)
            inv = 1.0 / den
            keep = eys != 0
            zero = jnp.zeros((_L,), jnp.float32)
            for r in range(R):
                sim_v[pl.ds(r * C, C)] = jnp.where(keep, es[r] * inv, zero)

            @plsc.parallel_loop(0, C, unroll=2)
            def out_body(n):
                j0 = n * R
                ws = [plsc.load_gather(
                          sim_v, [jnp.full((_L,), r * C, jnp.int32) + n])
                      for r in range(R)]
                for k in range(KT):
                    acc = _tree_sum(
                        [ws[r] * trow_v[p, j0 + r, pl.ds(k * _L, _L)]
                         for r in range(R)])
                    out_v[p, n, pl.ds(k * _L, _L)] = acc

            pltpu.sync_copy(focal_v.at[p],
                            out_hbm.at[pl.ds(base, C), pl.ds(0, D)])
            pltpu.sync_copy(out_v.at[p],
                            out_hbm.at[pl.ds(base, C), pl.ds(D, Td)])
            return carry

        lax.fori_loop(0, CH, chunk_body, 0)

    return sc_fn


def kernel(title_emb_mat, emp_ids, end_yrs, batch_label, inputs,
           ref_batch_pos, ref_job_idx, ref_title_idx):
    B, P, D = inputs.shape
    T, Td = title_emb_mat.shape
    R = ref_batch_pos.shape[-1]
    N = B * P

    inp_flat = inputs.reshape(N, D)
    pos_rows = ref_batch_pos.astype(jnp.int32).reshape(-1, 128)
    job_rows = ref_job_idx.astype(jnp.int32).reshape(-1, 128)
    tit_rows = ref_title_idx.astype(jnp.int32).reshape(-1, 128)
    ey_flat = end_yrs.astype(jnp.int32).reshape(-1)

    fn = _build_sc_call(N, D, Td, P, R, C=128 // R)
    return fn(inp_flat, title_emb_mat, pos_rows, job_rows, tit_rows, ey_flat)


# speedup vs baseline: 18.6652x; 1.2869x over previous
import functools

import jax
import jax.numpy as jnp
from jax import lax
from jax.experimental import pallas as pl
from jax.experimental.pallas import tpu as pltpu
from jax.experimental.pallas import tpu_sc as plsc

_NC = 2
_NS = 16
_NW = _NC * _NS
_L = 16


def _lane_sum(x):
    for m in (1, 2, 4, 8):
        perm = lax.iota(jnp.int32, _L) ^ m
        x = x + jnp.take_along_axis(x, perm, axis=0)
    return x


def _tree_sum(terms):
    while len(terms) > 1:
        nxt = [terms[i] + terms[i + 1] for i in range(0, len(terms) - 1, 2)]
        if len(terms) % 2:
            nxt.append(terms[-1])
        terms = nxt
    return terms[0]


def _build_sc_call(N, D, Td, P, R, C):
    rows_per_w = N // _NW
    CH = rows_per_w // C
    KD = D // _L
    KT = Td // _L
    G = C * R
    assert G == 128
    IW = rows_per_w * R // 128

    mesh = plsc.VectorSubcoreMesh(core_axis_name="c", subcore_axis_name="s")

    @functools.partial(
        pl.kernel,
        mesh=mesh,
        out_type=jax.ShapeDtypeStruct((N, D + Td), jnp.float32),
        compiler_params=pltpu.CompilerParams(needs_layout_passes=False),
        scratch_types=[
            pltpu.VMEM((IW, 128), jnp.int32),
            pltpu.VMEM((IW, 128), jnp.int32),
            pltpu.VMEM((IW, 128), jnp.int32),
            pltpu.VMEM((IW, 128), jnp.int32),
            pltpu.VMEM((2, C, D), jnp.float32),
            pltpu.VMEM((2, G, D), jnp.float32),
            pltpu.VMEM((2, G, Td), jnp.float32),
            pltpu.VMEM((G,), jnp.float32),
            pltpu.VMEM((G,), jnp.float32),
            pltpu.VMEM((rows_per_w,), jnp.int32),
            pltpu.VMEM((2, C, Td), jnp.float32),
            pltpu.SemaphoreType.DMA((2,)),
        ],
    )
    def sc_fn(inp_hbm, temb_hbm, pos_hbm, job_hbm, tit_hbm, ey_hbm, out_hbm,
              pos_v, job_v, fidx_v, tidx_v, focal_v, ref_v, trow_v,
              s_v, sim_v, ey_v, out_v, sem_in):
        wid = lax.axis_index("s") * _NC + lax.axis_index("c")
        base0 = wid * rows_per_w

        pltpu.sync_copy(pos_hbm.at[pl.ds(wid * IW, IW)], pos_v)
        pltpu.sync_copy(job_hbm.at[pl.ds(wid * IW, IW)], job_v)
        pltpu.sync_copy(tit_hbm.at[pl.ds(wid * IW, IW)], tidx_v)
        pltpu.sync_copy(ey_hbm.at[pl.ds(base0, rows_per_w)], ey_v)

        def flat_body(row, carry):
            for cc in range(128 // _L):
                f = (pos_v[row, pl.ds(cc * _L, _L)] * P
                     + job_v[row, pl.ds(cc * _L, _L)])
                fidx_v[row, pl.ds(cc * _L, _L)] = f
            return carry
        lax.fori_loop(0, IW, flat_body, 0)

        def issue(ci, p):
            base = base0 + ci * C
            pltpu.async_copy(inp_hbm.at[fidx_v.at[ci]], ref_v.at[p],
                             sem_in.at[p])
            pltpu.async_copy(temb_hbm.at[tidx_v.at[ci]], trow_v.at[p],
                             sem_in.at[p])
            pltpu.async_copy(inp_hbm.at[pl.ds(base, C)], focal_v.at[p],
                             sem_in.at[p])

        def drain(ci, p):
            base = base0 + ci * C
            pltpu.make_async_copy(inp_hbm.at[fidx_v.at[ci]], ref_v.at[p],
                                  sem_in.at[p]).wait()
            pltpu.make_async_copy(temb_hbm.at[tidx_v.at[ci]], trow_v.at[p],
                                  sem_in.at[p]).wait()
            pltpu.make_async_copy(inp_hbm.at[pl.ds(base, C)], focal_v.at[p],
                                  sem_in.at[p]).wait()

        issue(0, 0)
        lane0 = lax.iota(jnp.int32, _L) == 0

        def chunk_body(ci, carry):
            p = ci & 1
            base = base0 + ci * C

            @pl.when(ci + 1 < CH)
            def _():
                issue(ci + 1, 1 - p)

            drain(ci, p)

            @plsc.parallel_loop(0, C, unroll=4)
            def dot_body(n):
                j0 = n * R
                fv = [focal_v[p, n, pl.ds(k * _L, _L)] for k in range(KD)]
                for r in range(R):
                    prods = [fv[k] * ref_v[p, j0 + r, pl.ds(k * _L, _L)]
                             for k in range(KD)]
                    dv = _lane_sum(_tree_sum(prods))
                    idx = jnp.full((_L,), r * C, jnp.int32) + n
                    plsc.store_scatter(s_v, [idx], dv, mask=lane0)

            eys = ey_v[pl.ds(ci * C, C)]
            es = [jnp.exp(s_v[pl.ds(r * C, C)]) for r in range(R)]
            den = _tree_sum(es)
            inv = 1.0 / den
            keep = eys != 0
            zero = jnp.zeros((_L,), jnp.float32)
            for r in range(R):
                sim_v[pl.ds(r * C, C)] = jnp.where(keep, es[r] * inv, zero)

            @plsc.parallel_loop(0, C, unroll=4)
            def out_body(n):
                j0 = n * R
                ws = [plsc.load_gather(
                          sim_v, [jnp.full((_L,), r * C, jnp.int32) + n])
                      for r in range(R)]
                for k in range(KT):
                    acc = _tree_sum(
                        [ws[r] * trow_v[p, j0 + r, pl.ds(k * _L, _L)]
                         for r in range(R)])
                    out_v[p, n, pl.ds(k * _L, _L)] = acc

            pltpu.sync_copy(focal_v.at[p],
                            out_hbm.at[pl.ds(base, C), pl.ds(0, D)])
            pltpu.sync_copy(out_v.at[p],
                            out_hbm.at[pl.ds(base, C), pl.ds(D, Td)])
            return carry

        lax.fori_loop(0, CH, chunk_body, 0)

    return sc_fn


def kernel(title_emb_mat, emp_ids, end_yrs, batch_label, inputs,
           ref_batch_pos, ref_job_idx, ref_title_idx):
    B, P, D = inputs.shape
    T, Td = title_emb_mat.shape
    R = ref_batch_pos.shape[-1]
    N = B * P

    inp_flat = inputs.reshape(N, D)
    pos_rows = ref_batch_pos.astype(jnp.int32).reshape(-1, 128)
    job_rows = ref_job_idx.astype(jnp.int32).reshape(-1, 128)
    tit_rows = ref_title_idx.astype(jnp.int32).reshape(-1, 128)
    ey_flat = end_yrs.astype(jnp.int32).reshape(-1)

    fn = _build_sc_call(N, D, Td, P, R, C=128 // R)
    return fn(inp_flat, title_emb_mat, pos_rows, job_rows, tit_rows, ey_flat)
